# baseline (speedup 1.0000x reference)
import functools

import jax
import jax.numpy as jnp
from jax import lax
from jax.experimental import pallas as pl
from jax.experimental.pallas import tpu as pltpu
from jax.experimental.pallas import tpu_sc as plsc

_SEQ = 2048
_HID = 2048
_INT = 1024
_NE = 8
_CAP = 256
_KB = 512


def _gelu(x):
    return 0.5 * x * (1.0 + lax.erf(x * 0.7071067811865476))


def _route_body(x_ref, gw_ref, idx_ref, score_ref):
    x = x_ref[...]
    gw = gw_ref[...]
    logits = jnp.dot(x, gw, preferred_element_type=jnp.float32)
    m = jnp.max(logits, axis=1, keepdims=True)
    ex = jnp.exp(logits - m)
    aff = ex / jnp.sum(ex, axis=1, keepdims=True)
    bits = lax.bitcast_convert_type(aff, jnp.int32)

    def bs_body(_, lohi):
        lo, hi = lohi
        mid = lo + (hi - lo + 1) // 2
        cnt = jnp.sum((bits >= mid).astype(jnp.int32), axis=0, keepdims=True)
        ok = cnt >= _CAP
        return jnp.where(ok, mid, lo), jnp.where(ok, hi, mid - 1)

    thr, _ = lax.fori_loop(
        0, 31, bs_body,
        (jnp.zeros((1, _NE), jnp.int32), jnp.full((1, _NE), 0x3F800001, jnp.int32)),
    )

    gt = bits > thr
    tie = bits == thr
    n_gt = jnp.sum(gt.astype(jnp.int32), axis=0, keepdims=True)
    need = (_CAP - n_gt).astype(jnp.float32)

    row = lax.broadcasted_iota(jnp.int32, (_SEQ, _SEQ), 0)
    col = lax.broadcasted_iota(jnp.int32, (_SEQ, _SEQ), 1)
    ltri = (col < row).astype(jnp.bfloat16)
    both = jnp.concatenate([gt.astype(jnp.bfloat16), tie.astype(jnp.bfloat16)], axis=1)
    ecs = jnp.dot(ltri, both, preferred_element_type=jnp.float32)
    ecs_gt, ecs_tie = ecs[:, :_NE], ecs[:, _NE:]
    sel = gt | (tie & (ecs_tie < need))
    pos = ecs_gt + jnp.minimum(ecs_tie, need)

    iota_c = lax.broadcasted_iota(jnp.int32, (1, _CAP), 1).astype(jnp.float32)
    iota_r = lax.broadcasted_iota(jnp.int32, (_SEQ, 1), 0).astype(jnp.float32)
    for e in range(_NE):
        sel_e = sel[:, e:e + 1]
        pos_e = pos[:, e:e + 1]
        mf = (sel_e & (pos_e == iota_c)).astype(jnp.float32)
        idx_ref[pl.ds(e, 1), :] = jnp.sum(mf * iota_r, axis=0, keepdims=True).astype(jnp.int32)
        score_ref[pl.ds(e, 1), :] = jnp.sum(mf * aff[:, e:e + 1], axis=0, keepdims=True)


def _route(x, gate_w):
    return pl.pallas_call(
        _route_body,
        out_shape=[
            jax.ShapeDtypeStruct((_NE, _CAP), jnp.int32),
            jax.ShapeDtypeStruct((_NE, _CAP), jnp.float32),
        ],
    )(x, gate_w)


def _sc_gather_body(x_hbm, idx_hbm, out_hbm, idx_v, rows_v, sem):
    wid = lax.axis_index("s") * 2 + lax.axis_index("c")
    for chunk in range(2):
        base = wid * 64 + chunk * 32
        pltpu.sync_copy(idx_hbm.at[pl.ds(base, 32)], idx_v)
        pltpu.async_copy(x_hbm.at[idx_v], rows_v, sem).wait()
        pltpu.sync_copy(rows_v, out_hbm.at[pl.ds(base, 32)])


def _sc_gather(x, idx_flat):
    mesh = plsc.VectorSubcoreMesh(core_axis_name="c", subcore_axis_name="s")
    return pl.kernel(
        _sc_gather_body,
        out_type=jax.ShapeDtypeStruct((_NE * _CAP, _HID), jnp.float32),
        mesh=mesh,
        scratch_types=[
            pltpu.VMEM((32,), jnp.int32),
            pltpu.VMEM((32, _HID), jnp.float32),
            pltpu.SemaphoreType.DMA,
        ],
    )(x, idx_flat)


def _emlp_body(xg_ref, gw_ref, uw_ref, dw_ref, st_ref, out_ref):
    k = pl.program_id(1)
    xb = xg_ref[...].astype(jnp.bfloat16)
    g = jnp.dot(xb, gw_ref[0].astype(jnp.bfloat16), preferred_element_type=jnp.float32)
    u = jnp.dot(xb, uw_ref[0].astype(jnp.bfloat16), preferred_element_type=jnp.float32)
    h = _gelu(g) * u
    y = jnp.dot(h.astype(jnp.bfloat16), dw_ref[0].astype(jnp.bfloat16),
                preferred_element_type=jnp.float32)
    y = y * st_ref[0]

    @pl.when(k == 0)
    def _():
        out_ref[...] = y

    @pl.when(k != 0)
    def _():
        out_ref[...] += y


def _expert_mlp(xg, gpw, upw, dpw, scores_t):
    nk = _INT // _KB
    return pl.pallas_call(
        _emlp_body,
        grid=(_NE, nk),
        in_specs=[
            pl.BlockSpec((_CAP, _HID), lambda e, k: (e, 0)),
            pl.BlockSpec((1, _HID, _KB), lambda e, k: (e, 0, k)),
            pl.BlockSpec((1, _HID, _KB), lambda e, k: (e, 0, k)),
            pl.BlockSpec((1, _KB, _HID), lambda e, k: (e, k, 0)),
            pl.BlockSpec((1, _CAP, 1), lambda e, k: (e, 0, 0)),
        ],
        out_specs=pl.BlockSpec((_CAP, _HID), lambda e, k: (e, 0)),
        out_shape=jax.ShapeDtypeStruct((_NE * _CAP, _HID), jnp.float32),
    )(xg, gpw, upw, dpw, scores_t)


def _shared_body(x_ref, sgw_ref, suw_ref, sdw_ref, w_ref, idx_ref, out_ref, mt_scr):
    i = pl.program_id(0)

    @pl.when(i == 0)
    def _():
        ioty = lax.broadcasted_iota(jnp.int32, (_SEQ, _CAP), 0)
        for e in range(_NE):
            mt_scr[:, pl.ds(e * _CAP, _CAP)] = (
                idx_ref[pl.ds(e, 1), :] == ioty).astype(jnp.bfloat16)

    xb = x_ref[...].astype(jnp.bfloat16)
    g = jnp.dot(xb, sgw_ref[...], preferred_element_type=jnp.float32)
    u = jnp.dot(xb, suw_ref[...], preferred_element_type=jnp.float32)
    h = _gelu(g) * u
    y = jnp.dot(h.astype(jnp.bfloat16), sdw_ref[...],
                preferred_element_type=jnp.float32)
    out_t = jnp.dot(mt_scr[...], w_ref[...].astype(jnp.bfloat16),
                    preferred_element_type=jnp.float32)
    out_ref[...] = y + out_t.T


def _shared_final(x, sgw, suw, sdw, weighted, idx2d):
    nb = _SEQ // _CAP
    return pl.pallas_call(
        _shared_body,
        grid=(nb,),
        in_specs=[
            pl.BlockSpec((_CAP, _HID), lambda i: (i, 0)),
            pl.BlockSpec((_HID, _INT), lambda i: (0, 0)),
            pl.BlockSpec((_HID, _INT), lambda i: (0, 0)),
            pl.BlockSpec((_INT, _HID), lambda i: (0, 0)),
            pl.BlockSpec((_NE * _CAP, _CAP), lambda i: (0, i)),
            pl.BlockSpec((_NE, _CAP), lambda i: (0, 0)),
        ],
        out_specs=pl.BlockSpec((_CAP, _HID), lambda i: (i, 0)),
        out_shape=jax.ShapeDtypeStruct((_SEQ, _HID), jnp.float32),
        scratch_shapes=[pltpu.VMEM((_SEQ, _NE * _CAP), jnp.bfloat16)],
    )(x, sgw, suw, sdw, weighted, idx2d)


def kernel(hidden_states, gate_w, gate_proj_w, up_proj_w, down_proj_w,
           shared_gw, shared_uw, shared_dw):
    x = hidden_states.reshape(_SEQ, _HID)
    idx8, sc8 = _route(x, gate_w)
    idx_flat = idx8.reshape(_NE * _CAP)
    scores_t = sc8.reshape(_NE, _CAP, 1)
    xg = _sc_gather(x, idx_flat)
    weighted = _expert_mlp(xg, gate_proj_w, up_proj_w, down_proj_w, scores_t)
    out = _shared_final(x, shared_gw.astype(jnp.bfloat16),
                        shared_uw.astype(jnp.bfloat16),
                        shared_dw.astype(jnp.bfloat16), weighted, idx8)
    return out.reshape(1, _SEQ, _HID)


# speedup vs baseline: 1.0411x; 1.0411x over previous
import functools

import jax
import jax.numpy as jnp
from jax import lax
from jax.experimental import pallas as pl
from jax.experimental.pallas import tpu as pltpu
from jax.experimental.pallas import tpu_sc as plsc

_SEQ = 2048
_HID = 2048
_INT = 1024
_NE = 8
_CAP = 256
_KB = 512


def _gelu(x):
    return 0.5 * x * (1.0 + lax.erf(x * 0.7071067811865476))


def _route_body(x_ref, gw_ref, idx_ref, score_ref):
    x = x_ref[...]
    gw = gw_ref[...]
    logits = jnp.dot(x, gw, preferred_element_type=jnp.float32)
    m = jnp.max(logits, axis=1, keepdims=True)
    ex = jnp.exp(logits - m)
    aff = ex / jnp.sum(ex, axis=1, keepdims=True)
    bits = lax.bitcast_convert_type(aff, jnp.int32)

    def bs_body(_, lohi):
        lo, hi = lohi
        mid = lo + (hi - lo + 1) // 2
        cnt = jnp.sum((bits >= mid).astype(jnp.int32), axis=0, keepdims=True)
        ok = cnt >= _CAP
        return jnp.where(ok, mid, lo), jnp.where(ok, hi, mid - 1)

    thr, _ = lax.fori_loop(
        0, 31, bs_body,
        (jnp.zeros((1, _NE), jnp.int32), jnp.full((1, _NE), 0x3F800001, jnp.int32)),
    )

    gt = bits > thr
    tie = bits == thr
    n_gt = jnp.sum(gt.astype(jnp.int32), axis=0, keepdims=True)
    need = (_CAP - n_gt).astype(jnp.float32)

    row = lax.broadcasted_iota(jnp.int32, (_SEQ, _SEQ), 0)
    col = lax.broadcasted_iota(jnp.int32, (_SEQ, _SEQ), 1)
    ltri = (col < row).astype(jnp.bfloat16)
    both = jnp.concatenate([gt.astype(jnp.bfloat16), tie.astype(jnp.bfloat16)], axis=1)
    ecs = jnp.dot(ltri, both, preferred_element_type=jnp.float32)
    ecs_gt, ecs_tie = ecs[:, :_NE], ecs[:, _NE:]
    sel = gt | (tie & (ecs_tie < need))
    pos = ecs_gt + jnp.minimum(ecs_tie, need)

    iota_c = lax.broadcasted_iota(jnp.int32, (1, _CAP), 1).astype(jnp.float32)
    iota_r = lax.broadcasted_iota(jnp.int32, (_SEQ, 1), 0).astype(jnp.float32)
    for e in range(_NE):
        sel_e = sel[:, e:e + 1]
        pos_e = pos[:, e:e + 1]
        mf = (sel_e & (pos_e == iota_c)).astype(jnp.float32)
        idx_ref[pl.ds(e, 1), :] = jnp.sum(mf * iota_r, axis=0, keepdims=True).astype(jnp.int32)
        score_ref[pl.ds(e, 1), :] = jnp.sum(mf * aff[:, e:e + 1], axis=0, keepdims=True)


def _route(x, gate_w):
    return pl.pallas_call(
        _route_body,
        out_shape=[
            jax.ShapeDtypeStruct((_NE, _CAP), jnp.int32),
            jax.ShapeDtypeStruct((_NE, _CAP), jnp.float32),
        ],
    )(x, gate_w)


def _sc_gather_body(x_hbm, idx_hbm, out_hbm, idx_v, rows_v, sem):
    wid = lax.axis_index("s") * 2 + lax.axis_index("c")
    for chunk in range(2):
        base = wid * 64 + chunk * 32
        pltpu.sync_copy(idx_hbm.at[pl.ds(base, 32)], idx_v)
        pltpu.async_copy(x_hbm.at[idx_v], rows_v, sem).wait()
        pltpu.sync_copy(rows_v, out_hbm.at[pl.ds(base, 32)])


def _sc_gather(x, idx_flat):
    mesh = plsc.VectorSubcoreMesh(core_axis_name="c", subcore_axis_name="s")
    return pl.kernel(
        _sc_gather_body,
        out_type=jax.ShapeDtypeStruct((_NE * _CAP, _HID), jnp.float32),
        mesh=mesh,
        scratch_types=[
            pltpu.VMEM((32,), jnp.int32),
            pltpu.VMEM((32, _HID), jnp.float32),
            pltpu.SemaphoreType.DMA,
        ],
    )(x, idx_flat)


def _emlp_body(xg_ref, gw_ref, uw_ref, dw_ref, st_ref, out_ref):
    k = pl.program_id(1)
    xb = xg_ref[...].astype(jnp.bfloat16)
    g = jnp.dot(xb, gw_ref[0].astype(jnp.bfloat16), preferred_element_type=jnp.float32)
    u = jnp.dot(xb, uw_ref[0].astype(jnp.bfloat16), preferred_element_type=jnp.float32)
    h = _gelu(g) * u
    y = jnp.dot(h.astype(jnp.bfloat16), dw_ref[0].astype(jnp.bfloat16),
                preferred_element_type=jnp.float32)
    y = y * st_ref[0]

    @pl.when(k == 0)
    def _():
        out_ref[...] = y

    @pl.when(k != 0)
    def _():
        out_ref[...] += y


def _expert_mlp(xg, gpw, upw, dpw, scores_t):
    nk = _INT // _KB
    return pl.pallas_call(
        _emlp_body,
        grid=(_NE, nk),
        in_specs=[
            pl.BlockSpec((_CAP, _HID), lambda e, k: (e, 0)),
            pl.BlockSpec((1, _HID, _KB), lambda e, k: (e, 0, k)),
            pl.BlockSpec((1, _HID, _KB), lambda e, k: (e, 0, k)),
            pl.BlockSpec((1, _KB, _HID), lambda e, k: (e, k, 0)),
            pl.BlockSpec((1, _CAP, 1), lambda e, k: (e, 0, 0)),
        ],
        out_specs=pl.BlockSpec((_CAP, _HID), lambda e, k: (e, 0)),
        out_shape=jax.ShapeDtypeStruct((_NE * _CAP, _HID), jnp.float32),
    )(xg, gpw, upw, dpw, scores_t)


def _shared_body(x_ref, sgw_ref, suw_ref, sdw_ref, w_ref, idx_ref, out_ref, mt_scr):
    i = pl.program_id(0)

    @pl.when(i == 0)
    def _():
        ioty = lax.broadcasted_iota(jnp.int32, (_SEQ, _CAP), 0)
        for e in range(_NE):
            mt_scr[:, pl.ds(e * _CAP, _CAP)] = (
                idx_ref[pl.ds(e, 1), :] == ioty).astype(jnp.bfloat16)

    xb = x_ref[...].astype(jnp.bfloat16)
    g = jnp.dot(xb, sgw_ref[...].astype(jnp.bfloat16), preferred_element_type=jnp.float32)
    u = jnp.dot(xb, suw_ref[...].astype(jnp.bfloat16), preferred_element_type=jnp.float32)
    h = _gelu(g) * u
    y = jnp.dot(h.astype(jnp.bfloat16), sdw_ref[...].astype(jnp.bfloat16),
                preferred_element_type=jnp.float32)
    out_t = jnp.dot(mt_scr[...], w_ref[...].astype(jnp.bfloat16),
                    preferred_element_type=jnp.float32)
    out_ref[...] = y + out_t.T


def _shared_final(x, sgw, suw, sdw, weighted, idx2d):
    nb = _SEQ // _CAP
    return pl.pallas_call(
        _shared_body,
        grid=(nb,),
        in_specs=[
            pl.BlockSpec((_CAP, _HID), lambda i: (i, 0)),
            pl.BlockSpec((_HID, _INT), lambda i: (0, 0)),
            pl.BlockSpec((_HID, _INT), lambda i: (0, 0)),
            pl.BlockSpec((_INT, _HID), lambda i: (0, 0)),
            pl.BlockSpec((_NE * _CAP, _CAP), lambda i: (0, i)),
            pl.BlockSpec((_NE, _CAP), lambda i: (0, 0)),
        ],
        out_specs=pl.BlockSpec((_CAP, _HID), lambda i: (i, 0)),
        out_shape=jax.ShapeDtypeStruct((_SEQ, _HID), jnp.float32),
        scratch_shapes=[pltpu.VMEM((_SEQ, _NE * _CAP), jnp.bfloat16)],
    )(x, sgw, suw, sdw, weighted, idx2d)


def kernel(hidden_states, gate_w, gate_proj_w, up_proj_w, down_proj_w,
           shared_gw, shared_uw, shared_dw):
    x = hidden_states.reshape(_SEQ, _HID)
    idx8, sc8 = _route(x, gate_w)
    idx_flat = idx8.reshape(_NE * _CAP)
    scores_t = sc8.reshape(_NE, _CAP, 1)
    xg = _sc_gather(x, idx_flat)
    weighted = _expert_mlp(xg, gate_proj_w, up_proj_w, down_proj_w, scores_t)
    out = _shared_final(x, shared_gw, shared_uw, shared_dw, weighted, idx8)
    return out.reshape(1, _SEQ, _HID)


# speedup vs baseline: 1.0487x; 1.0073x over previous
import functools

import jax
import jax.numpy as jnp
from jax import lax
from jax.experimental import pallas as pl
from jax.experimental.pallas import tpu as pltpu
from jax.experimental.pallas import tpu_sc as plsc

_SEQ = 2048
_HID = 2048
_INT = 1024
_NE = 8
_CAP = 256
_KB = 512


def _gelu(x):
    return 0.5 * x * (1.0 + lax.erf(x * 0.7071067811865476))


def _route_body(x_ref, gw_ref, idx_ref, score_ref):
    x = x_ref[...]
    gw = gw_ref[...]
    logits = jnp.dot(x, gw, preferred_element_type=jnp.float32)
    m = jnp.max(logits, axis=1, keepdims=True)
    ex = jnp.exp(logits - m)
    aff = ex / jnp.sum(ex, axis=1, keepdims=True)
    bits = lax.bitcast_convert_type(aff, jnp.int32)

    def bs_body(_, lohi):
        lo, hi = lohi
        mid = lo + (hi - lo + 1) // 2
        cnt = jnp.sum((bits >= mid).astype(jnp.int32), axis=0, keepdims=True)
        ok = cnt >= _CAP
        return jnp.where(ok, mid, lo), jnp.where(ok, hi, mid - 1)

    thr, _ = lax.fori_loop(
        0, 31, bs_body,
        (jnp.zeros((1, _NE), jnp.int32), jnp.full((1, _NE), 0x3F800001, jnp.int32)),
    )

    gt = bits > thr
    tie = bits == thr
    n_gt = jnp.sum(gt.astype(jnp.int32), axis=0, keepdims=True)
    need = (_CAP - n_gt).astype(jnp.float32)

    row = lax.broadcasted_iota(jnp.int32, (_SEQ, 1), 0)
    col = lax.broadcasted_iota(jnp.int32, (1, _SEQ), 1)
    ltri = (col < row).astype(jnp.bfloat16)
    both = jnp.concatenate([gt.astype(jnp.bfloat16), tie.astype(jnp.bfloat16)], axis=1)
    ecs = jnp.dot(ltri, both, preferred_element_type=jnp.float32)
    ecs_gt, ecs_tie = ecs[:, :_NE], ecs[:, _NE:]
    sel = gt | (tie & (ecs_tie < need))
    pos = ecs_gt + jnp.minimum(ecs_tie, need)

    iota_c = lax.broadcasted_iota(jnp.int32, (1, _CAP), 1).astype(jnp.float32)
    iota_r = lax.broadcasted_iota(jnp.int32, (_SEQ, 1), 0).astype(jnp.float32)
    for e in range(_NE):
        sel_e = sel[:, e:e + 1]
        pos_e = pos[:, e:e + 1]
        mf = (sel_e & (pos_e == iota_c)).astype(jnp.float32)
        idx_ref[pl.ds(e, 1), :] = jnp.sum(mf * iota_r, axis=0, keepdims=True).astype(jnp.int32)
        score_ref[pl.ds(e, 1), :] = jnp.sum(mf * aff[:, e:e + 1], axis=0, keepdims=True)


def _route(x, gate_w):
    return pl.pallas_call(
        _route_body,
        out_shape=[
            jax.ShapeDtypeStruct((_NE, _CAP), jnp.int32),
            jax.ShapeDtypeStruct((_NE, _CAP), jnp.float32),
        ],
    )(x, gate_w)


def _sc_gather_body(x_hbm, idx_hbm, out_hbm, idx_v, rows_v, sem):
    wid = lax.axis_index("s") * 2 + lax.axis_index("c")
    for chunk in range(2):
        base = wid * 64 + chunk * 32
        pltpu.sync_copy(idx_hbm.at[pl.ds(base, 32)], idx_v)
        pltpu.async_copy(x_hbm.at[idx_v], rows_v, sem).wait()
        pltpu.sync_copy(rows_v, out_hbm.at[pl.ds(base, 32)])


def _sc_gather(x, idx_flat):
    mesh = plsc.VectorSubcoreMesh(core_axis_name="c", subcore_axis_name="s")
    return pl.kernel(
        _sc_gather_body,
        out_type=jax.ShapeDtypeStruct((_NE * _CAP, _HID), jnp.float32),
        mesh=mesh,
        scratch_types=[
            pltpu.VMEM((32,), jnp.int32),
            pltpu.VMEM((32, _HID), jnp.float32),
            pltpu.SemaphoreType.DMA,
        ],
    )(x, idx_flat)


def _emlp_body(xg_ref, gw_ref, uw_ref, dw_ref, st_ref, out_ref):
    k = pl.program_id(1)
    xb = xg_ref[...].astype(jnp.bfloat16)
    g = jnp.dot(xb, gw_ref[0].astype(jnp.bfloat16), preferred_element_type=jnp.float32)
    u = jnp.dot(xb, uw_ref[0].astype(jnp.bfloat16), preferred_element_type=jnp.float32)
    h = _gelu(g) * u
    y = jnp.dot(h.astype(jnp.bfloat16), dw_ref[0].astype(jnp.bfloat16),
                preferred_element_type=jnp.float32)
    y = y * st_ref[0]

    @pl.when(k == 0)
    def _():
        out_ref[...] = y

    @pl.when(k != 0)
    def _():
        out_ref[...] += y


def _expert_mlp(xg, gpw, upw, dpw, scores_t):
    nk = _INT // _KB
    return pl.pallas_call(
        _emlp_body,
        grid=(_NE, nk),
        in_specs=[
            pl.BlockSpec((_CAP, _HID), lambda e, k: (e, 0)),
            pl.BlockSpec((1, _HID, _KB), lambda e, k: (e, 0, k)),
            pl.BlockSpec((1, _HID, _KB), lambda e, k: (e, 0, k)),
            pl.BlockSpec((1, _KB, _HID), lambda e, k: (e, k, 0)),
            pl.BlockSpec((1, _CAP, 1), lambda e, k: (e, 0, 0)),
        ],
        out_specs=pl.BlockSpec((_CAP, _HID), lambda e, k: (e, 0)),
        out_shape=jax.ShapeDtypeStruct((_NE * _CAP, _HID), jnp.float32),
    )(xg, gpw, upw, dpw, scores_t)


def _shared_body(x_ref, sgw_ref, suw_ref, sdw_ref, w_ref, idx_ref, out_ref, mt_scr):
    i = pl.program_id(0)

    @pl.when(i == 0)
    def _():
        ioty = lax.broadcasted_iota(jnp.int32, (_SEQ, _CAP), 0)
        for e in range(_NE):
            mt_scr[:, pl.ds(e * _CAP, _CAP)] = (
                idx_ref[pl.ds(e, 1), :] == ioty).astype(jnp.bfloat16)

    xb = x_ref[...].astype(jnp.bfloat16)
    g = jnp.dot(xb, sgw_ref[...].astype(jnp.bfloat16), preferred_element_type=jnp.float32)
    u = jnp.dot(xb, suw_ref[...].astype(jnp.bfloat16), preferred_element_type=jnp.float32)
    h = _gelu(g) * u
    y = jnp.dot(h.astype(jnp.bfloat16), sdw_ref[...].astype(jnp.bfloat16),
                preferred_element_type=jnp.float32)
    out_t = jnp.dot(mt_scr[...], w_ref[...].astype(jnp.bfloat16),
                    preferred_element_type=jnp.float32)
    out_ref[...] = y + out_t.T


def _shared_final(x, sgw, suw, sdw, weighted, idx2d):
    nb = _SEQ // _CAP
    return pl.pallas_call(
        _shared_body,
        grid=(nb,),
        in_specs=[
            pl.BlockSpec((_CAP, _HID), lambda i: (i, 0)),
            pl.BlockSpec((_HID, _INT), lambda i: (0, 0)),
            pl.BlockSpec((_HID, _INT), lambda i: (0, 0)),
            pl.BlockSpec((_INT, _HID), lambda i: (0, 0)),
            pl.BlockSpec((_NE * _CAP, _CAP), lambda i: (0, i)),
            pl.BlockSpec((_NE, _CAP), lambda i: (0, 0)),
        ],
        out_specs=pl.BlockSpec((_CAP, _HID), lambda i: (i, 0)),
        out_shape=jax.ShapeDtypeStruct((_SEQ, _HID), jnp.float32),
        scratch_shapes=[pltpu.VMEM((_SEQ, _NE * _CAP), jnp.bfloat16)],
    )(x, sgw, suw, sdw, weighted, idx2d)


def kernel(hidden_states, gate_w, gate_proj_w, up_proj_w, down_proj_w,
           shared_gw, shared_uw, shared_dw):
    x = hidden_states.reshape(_SEQ, _HID)
    idx8, sc8 = _route(x, gate_w)
    idx_flat = idx8.reshape(_NE * _CAP)
    scores_t = sc8.reshape(_NE, _CAP, 1)
    xg = _sc_gather(x, idx_flat)
    weighted = _expert_mlp(xg, gate_proj_w, up_proj_w, down_proj_w, scores_t)
    out = _shared_final(x, shared_gw, shared_uw, shared_dw, weighted, idx8)
    return out.reshape(1, _SEQ, _HID)
